# baseline (speedup 1.0000x reference)
import functools

import jax
import jax.numpy as jnp
from jax.experimental import pallas as pl
from jax.experimental.pallas import tpu as pltpu
from jax.experimental.pallas import tpu_sc as plsc

_WINDOW = 128
_LANES = 16


def kernel(x, weight):
    rows, cols = x.shape
    num_idx = rows * cols
    vocab, dim = weight.shape
    nwin = num_idx // _WINDOW
    xf = x.reshape(nwin, _WINDOW).astype(jnp.int32)
    qidx = xf >> 1
    hbase = (xf & 1) * dim
    w2 = weight.reshape(vocab // 2, 2 * dim)

    mesh = plsc.VectorSubcoreMesh(core_axis_name="c", subcore_axis_name="s")

    @functools.partial(
        pl.kernel,
        out_type=jax.ShapeDtypeStruct((num_idx * dim // 128, 128), weight.dtype),
        mesh=mesh,
        scratch_types=[pltpu.VMEM((_WINDOW, 2 * dim), weight.dtype)],
        compiler_params=pltpu.CompilerParams(use_tc_tiling_on_sc=False),
    )
    def gather_kernel(w_hbm, q_hbm, h_hbm, o_hbm, pairs_ref):
        def body(q_vmem, h_vmem, o_vmem):
            pltpu.sync_copy(w_hbm.at[q_vmem.at[0]], pairs_ref)

            @pl.loop(0, _WINDOW // _LANES)
            def _(k):
                hv = h_vmem[0, pl.ds(k * _LANES, _LANES)]
                for j in range(_LANES // 2):
                    m = k * (_LANES // 2) + j
                    b_lo = hv[2 * j]
                    b_hi = hv[2 * j + 1]
                    for c in range(0, dim, _LANES):
                        o_vmem[m, pl.ds(c, _LANES)] = pairs_ref[
                            2 * m, pl.ds(b_lo + c, _LANES)
                        ]
                        o_vmem[m, pl.ds(dim + c, _LANES)] = pairs_ref[
                            2 * m + 1, pl.ds(b_hi + c, _LANES)
                        ]

        pltpu.emit_pipeline(
            body,
            grid=(nwin,),
            in_specs=[
                pl.BlockSpec((1, _WINDOW), lambda i: (i, 0)),
                pl.BlockSpec((1, _WINDOW), lambda i: (i, 0)),
            ],
            out_specs=[
                pl.BlockSpec((_WINDOW * dim // 128, 128), lambda i: (i, 0))
            ],
            core_axis_name=("c", "s"),
            dimension_semantics=(pltpu.PARALLEL,),
        )(q_hbm, h_hbm, o_hbm)

    out = gather_kernel(w2, qidx, hbase)
    return out.reshape(rows, cols, dim)
